# Initial kernel scaffold; baseline (speedup 1.0000x reference)
import functools

import jax
import jax.numpy as jnp
from jax import lax
from jax.experimental import pallas as pl
from jax.experimental.pallas import tpu as pltpu
from jax.experimental.pallas import tpu_sc as plsc

NC = 2
NS = 16
NW = NC * NS
LANES = 16


@functools.lru_cache(maxsize=None)
def _build(N, K, V, D, P):
    assert N % NW == 0
    per_w = N // NW
    assert per_w % P == 0
    n_chunks = per_w // P
    mesh = plsc.VectorSubcoreMesh(core_axis_name="c", subcore_axis_name="s")

    @functools.partial(
        pl.kernel,
        mesh=mesh,
        out_type=jax.ShapeDtypeStruct((N, D), jnp.float32),
        scratch_types=[
            pltpu.VMEM((K * P,), jnp.int32),
            pltpu.VMEM((K * P, D), jnp.float32),
            pltpu.VMEM((P, D), jnp.float32),
            pltpu.SemaphoreType.DMA,
        ],
    )
    def k(idx_hbm, table_hbm, out_hbm, idx_v, rows_v, out_v, sem):
        wid = lax.axis_index("s") * NC + lax.axis_index("c")
        base_w = wid * per_w

        def chunk_body(c, _):
            base = base_w + c * P
            pltpu.sync_copy(idx_hbm.at[pl.ds(base * K, K * P)], idx_v)
            pltpu.async_copy(table_hbm.at[idx_v], rows_v, sem).wait()

            def pos_body(p, _):
                r = K * p
                for half in range(D // LANES):
                    off = half * LANES
                    acc = rows_v[r, pl.ds(off, LANES)]
                    for kk in range(1, K):
                        acc = acc + rows_v[r + kk, pl.ds(off, LANES)]
                    out_v[p, pl.ds(off, LANES)] = acc
                return 0

            lax.fori_loop(0, P, pos_body, 0)
            pltpu.sync_copy(out_v, out_hbm.at[pl.ds(base, P)])
            return 0

        lax.fori_loop(0, n_chunks, chunk_body, 0)

    return k


def kernel(word_inputs, word_seq_lengths, word_embedding_table):
    B, L, K = word_inputs.shape
    V, D = word_embedding_table.shape
    N = B * L
    idx_flat = word_inputs.astype(jnp.int32).reshape(N * K)
    k = _build(N, K, V, D, 640)
    out = k(idx_flat, word_embedding_table)
    return out.reshape(B, L, D)


# speedup vs baseline: 1.6915x; 1.6915x over previous
import functools

import jax
import jax.numpy as jnp
from jax import lax
from jax.experimental import pallas as pl
from jax.experimental.pallas import tpu as pltpu
from jax.experimental.pallas import tpu_sc as plsc

NC = 2
NS = 16
NW = NC * NS
LANES = 16


@functools.lru_cache(maxsize=None)
def _build(N, K, V, D, P):
    assert N % NW == 0
    per_w = N // NW
    assert per_w % P == 0
    n_chunks = per_w // P
    mesh = plsc.VectorSubcoreMesh(core_axis_name="c", subcore_axis_name="s")

    @functools.partial(
        pl.kernel,
        mesh=mesh,
        compiler_params=pltpu.CompilerParams(use_tc_tiling_on_sc=False),
        out_type=jax.ShapeDtypeStruct((N, D), jnp.float32),
        scratch_types=[
            pltpu.VMEM((K * P,), jnp.int32),
            pltpu.VMEM((K * P, D), jnp.float32),
            pltpu.VMEM((P, D), jnp.float32),
            pltpu.SemaphoreType.DMA,
        ],
    )
    def k(idx_hbm, table_hbm, out_hbm, idx_v, rows_v, out_v, sem):
        wid = lax.axis_index("s") * NC + lax.axis_index("c")
        base_w = wid * per_w

        def chunk_body(c, _):
            base = base_w + c * P
            pltpu.sync_copy(idx_hbm.at[pl.ds(base * K, K * P)], idx_v)
            pltpu.async_copy(table_hbm.at[idx_v], rows_v, sem).wait()

            def pos_body(p, _):
                r = K * p
                for half in range(D // LANES):
                    off = half * LANES
                    acc = rows_v[r, pl.ds(off, LANES)]
                    for kk in range(1, K):
                        acc = acc + rows_v[r + kk, pl.ds(off, LANES)]
                    out_v[p, pl.ds(off, LANES)] = acc
                return 0

            lax.fori_loop(0, P, pos_body, 0)
            pltpu.sync_copy(out_v, out_hbm.at[pl.ds(base, P)])
            return 0

        lax.fori_loop(0, n_chunks, chunk_body, 0)

    return k


def kernel(word_inputs, word_seq_lengths, word_embedding_table):
    B, L, K = word_inputs.shape
    V, D = word_embedding_table.shape
    N = B * L
    idx_flat = word_inputs.astype(jnp.int32).reshape(N * K)
    k = _build(N, K, V, D, 640)
    out = k(idx_flat, word_embedding_table)
    return out.reshape(B, L, D)


# speedup vs baseline: 2.3923x; 1.4143x over previous
import functools

import jax
import jax.numpy as jnp
from jax import lax
from jax.experimental import pallas as pl
from jax.experimental.pallas import tpu as pltpu
from jax.experimental.pallas import tpu_sc as plsc

NC = 2
NS = 16
NW = NC * NS
LANES = 16

B, L, K, D = 4096, 50, 4, 32
BW = B // NW
NIDX = K * BW


def _make_kernel(V):
    mesh = plsc.VectorSubcoreMesh(core_axis_name="c", subcore_axis_name="s")

    @functools.partial(
        pl.kernel,
        mesh=mesh,
        compiler_params=pltpu.CompilerParams(use_tc_tiling_on_sc=False),
        out_type=jax.ShapeDtypeStruct((L, NW, BW, D), jnp.float32),
        scratch_types=[
            pltpu.VMEM((NIDX,), jnp.int32),
            pltpu.VMEM((NIDX,), jnp.int32),
            pltpu.VMEM((NIDX, D), jnp.float32),
            pltpu.VMEM((NIDX, D), jnp.float32),
            pltpu.VMEM((BW, D), jnp.float32),
            pltpu.VMEM((BW, D), jnp.float32),
            pltpu.SemaphoreType.DMA,
            pltpu.SemaphoreType.DMA,
            pltpu.SemaphoreType.DMA,
        ],
    )
    def k(idx_hbm, table_hbm, out_hbm, x0, x1, r0, r1, o0, o1,
          isem, gsem, osem):
        wid = lax.axis_index("s") * NC + lax.axis_index("c")
        xs, rs, os_ = (x0, x1), (r0, r1), (o0, o1)

        def idx_dma(l):
            return pltpu.make_async_copy(
                idx_hbm.at[l, wid], xs[l % 2], isem)

        def gather_dma(l):
            return pltpu.make_async_copy(
                table_hbm.at[xs[l % 2]], rs[l % 2], gsem)

        def out_dma(l):
            return pltpu.make_async_copy(
                os_[l % 2], out_hbm.at[l, wid], osem)

        def compute(l):
            r, o = rs[l % 2], os_[l % 2]

            def bo_body(bo, _):
                for half in range(D // LANES):
                    sl = pl.ds(half * LANES, LANES)
                    acc = r[bo, sl]
                    for kk in range(1, K):
                        acc = acc + r[kk * BW + bo, sl]
                    o[bo, sl] = acc
                return 0

            lax.fori_loop(0, BW, bo_body, 0)

        idx_dma(0).start()
        idx_dma(0).wait()
        gather_dma(0).start()
        idx_dma(1).start()
        pending_out = []
        for l in range(L):
            gather_dma(l).wait()
            if l + 2 < L:
                idx_dma(l + 2).start()
            if l + 1 < L:
                idx_dma(l + 1).wait()
                gather_dma(l + 1).start()
            if len(pending_out) == 2:
                pending_out.pop(0).wait()
            compute(l)
            dma = out_dma(l)
            dma.start()
            pending_out.append(dma)
        for dma in pending_out:
            dma.wait()

    return k


def kernel(word_inputs, word_seq_lengths, word_embedding_table):
    V, _ = word_embedding_table.shape
    idx5 = (word_inputs.astype(jnp.int32)
            .reshape(NW, BW, L, K)
            .transpose(2, 0, 3, 1)
            .reshape(L, NW, NIDX))
    out = _make_kernel(V)(idx5, word_embedding_table)
    return out.reshape(L, B, D).transpose(1, 0, 2)
